# baseline (speedup 1.0000x reference)
import jax
import jax.numpy as jnp
from jax import lax
from jax.experimental import pallas as pl
from jax.experimental.pallas import tpu as pltpu
from jax.experimental.pallas import tpu_sc as plsc

_N = 10000
_D = 128
_G = 64
_NC = 2
_NS = 16
_NW = _NC * _NS
_K = 128
_NP = 10240
_RPS = _NP // _NS


def _ceil_div(a, b):
    return -(-a // b)


def _zero_fill(ref, value):
    def fill(t, carry):
        ref[t // 8, pl.ds((t % 8) * 16, 16)] = jnp.full((16,), value, jnp.float32)
        return carry

    lax.fori_loop(0, _K * (_D // 16), fill, 0)


def _scatter_call(hp, srcp, dstp):
    nch = srcp.shape[1]
    assert nch % 4 == 0
    hch = nch // 2

    def body(hp_hbm, src_hbm, dst_hbm, out_hbm, srcb, dstb, gb0, gb1, acc,
             sem0, sem1):
        c = lax.axis_index("c")
        s = lax.axis_index("s")
        wid = c * _NS + s

        _zero_fill(gb0, 0.0)
        base = s * _RPS
        for r in range(_RPS // _K):
            pltpu.sync_copy(gb0, acc.at[pl.ds(base + r * _K, _K)])

        for h in range(2):
            pltpu.sync_copy(src_hbm.at[wid, pl.ds(h * hch, hch)], srcb)
            pltpu.sync_copy(dst_hbm.at[wid, pl.ds(h * hch, hch)], dstb)
            if h == 0:
                plsc.subcore_barrier()
            pltpu.async_copy(hp_hbm.at[srcb.at[0]], gb0, sem0)

            def step(p, carry):
                j = 2 * p
                pltpu.make_async_copy(hp_hbm.at[srcb.at[0]], gb0, sem0).wait()
                pltpu.async_copy(hp_hbm.at[srcb.at[j + 1]], gb1, sem1)
                pltpu.sync_copy(gb0, acc.at[dstb.at[j]], add=True)
                pltpu.make_async_copy(hp_hbm.at[srcb.at[0]], gb1, sem1).wait()
                nxt = jnp.minimum(j + 2, hch - 1)
                pltpu.async_copy(hp_hbm.at[srcb.at[nxt]], gb0, sem0)
                pltpu.sync_copy(gb1, acc.at[dstb.at[j + 1]], add=True)
                return carry

            lax.fori_loop(0, hch // 2, step, 0)
            pltpu.make_async_copy(hp_hbm.at[srcb.at[0]], gb0, sem0).wait()

        plsc.subcore_barrier()
        pltpu.sync_copy(acc.at[pl.ds(base, _RPS)],
                        out_hbm.at[c, pl.ds(base, _RPS)])

    return pl.kernel(
        body,
        out_type=jax.ShapeDtypeStruct((_NC, _NP, _D), jnp.float32),
        mesh=plsc.VectorSubcoreMesh(core_axis_name="c", subcore_axis_name="s"),
        scratch_types=[
            pltpu.VMEM((nch // 2, _K), jnp.int32),
            pltpu.VMEM((nch // 2, _K), jnp.int32),
            pltpu.VMEM((_K, _D), jnp.float32),
            pltpu.VMEM((_K, _D), jnp.float32),
            pltpu.VMEM_SHARED((_NP, _D), jnp.float32),
            pltpu.SemaphoreType.DMA,
            pltpu.SemaphoreType.DMA,
        ],
    )(hp, srcp, dstp)


def _rsqrt(v):
    r = lax.rsqrt(v)
    r = r * (1.5 - 0.5 * v * r * r)
    r = r * (1.5 - 0.5 * v * r * r)
    return r


def _colmean(v):
    n = v.shape[0]
    ones_row = jnp.ones((1, n), jnp.float32)
    return lax.dot_general(ones_row, v, (((1,), (0,)), ((), ())),
                           preferred_element_type=jnp.float32,
                           precision=lax.Precision.HIGHEST) * (1.0 / n)


def _bn_relu(v, gamma, beta):
    mu = _colmean(v)
    var = _colmean((v - mu) ** 2)
    return jnp.maximum((v - mu) * _rsqrt(var + 1e-5) * gamma + beta, 0.0)


def _tc1_body(x_ref, w_ref, dpt_ref, h_ref, dinv_ref):
    dpt = dpt_ref[...]
    deg = dpt[:_N, 0:1] + dpt[:_N, 1:2] + 1.0
    dinv = _rsqrt(deg)
    h = jnp.dot(x_ref[...], w_ref[...], preferred_element_type=jnp.float32)
    h_ref[pl.ds(0, _N), :] = h * dinv
    h_ref[pl.ds(_N, _NP - _N), :] = jnp.zeros((_NP - _N, _D), jnp.float32)
    dinv_ref[...] = dinv


def _tc1_call(x, W1, dpt):
    return pl.pallas_call(
        _tc1_body,
        out_shape=(jax.ShapeDtypeStruct((_NP, _D), jnp.float32),
                   jax.ShapeDtypeStruct((_N, 1), jnp.float32)),
    )(x, W1, dpt)


def _tc2_body(p_ref, hp_ref, dinv_ref, b_ref, g_ref, be_ref, w_ref, out_ref):
    tot = p_ref[0] + p_ref[1] + hp_ref[...]
    dinv = dinv_ref[...]
    conv = tot[:_N] * dinv + b_ref[...]
    a = _bn_relu(conv, g_ref[...], be_ref[...])
    h2 = jnp.dot(a, w_ref[...], preferred_element_type=jnp.float32)
    out_ref[pl.ds(0, _N), :] = h2 * dinv
    out_ref[pl.ds(_N, _NP - _N), :] = jnp.zeros((_NP - _N, _D), jnp.float32)


def _tc2_call(p, hp, dinv, b1, g1, be1, W2):
    return pl.pallas_call(
        _tc2_body,
        out_shape=jax.ShapeDtypeStruct((_NP, _D), jnp.float32),
    )(p, hp, dinv, b1, g1, be1, W2)


def _tc3_body(p_ref, hp_ref, dinv_ref, b_ref, g_ref, be_ref, bt_ref,
              w1_ref, b1_ref, g1_ref, be1_ref,
              w2_ref, b2_ref, g2_ref, be2_ref,
              w3_ref, b3_ref, w4_ref, b4_ref, out_ref):
    tot = p_ref[0] + p_ref[1] + hp_ref[...]
    conv = tot[:_N] * dinv_ref[...] + b_ref[...]
    a = _bn_relu(conv, g_ref[...], be_ref[...])
    gid = lax.broadcasted_iota(jnp.int32, (_N, _G), 1)
    onehot = (bt_ref[...] == gid).astype(jnp.float32)
    hi = lax.Precision.HIGHEST
    sums = lax.dot_general(onehot, a, (((0,), (0,)), ((), ())),
                           preferred_element_type=jnp.float32,
                           precision=hi)
    cnt = lax.dot_general(onehot, jnp.ones((_N, 1), jnp.float32),
                          (((0,), (0,)), ((), ())),
                          preferred_element_type=jnp.float32,
                          precision=hi)
    pooled = sums / jnp.maximum(cnt, 1.0)
    z = _bn_relu(jnp.dot(pooled, w1_ref[...]) + b1_ref[...],
                 g1_ref[...], be1_ref[...])
    z = _bn_relu(jnp.dot(z, w2_ref[...]) + b2_ref[...],
                 g2_ref[...], be2_ref[...])
    z = jnp.maximum(jnp.dot(z, w3_ref[...]) + b3_ref[...], 0.0)
    out_ref[...] = jnp.dot(z, w4_ref[...]) + b4_ref[...]


def _tc3_call(p, hp, dinv, b2, g2, be2, batch2d,
              Wo1, bo1, go1, beo1, Wo2, bo2, go2, beo2, Wo3, bo3, Wo4, bo4):
    return pl.pallas_call(
        _tc3_body,
        out_shape=jax.ShapeDtypeStruct((_G, 1), jnp.float32),
    )(p, hp, dinv, b2, g2, be2, batch2d,
      Wo1, bo1, go1, beo1, Wo2, bo2, go2, beo2, Wo3, bo3, Wo4, bo4)


def kernel(x, edge_index, batch,
           W1, b1, g1, be1, W2, b2, g2, be2,
           Wo1, bo1, go1, beo1, Wo2, bo2, go2, beo2, Wo3, bo3, Wo4, bo4):
    e = edge_index.shape[1]
    nch = 4 * _ceil_div(e, _NW * _K * 4)
    epad = _NW * nch * _K
    padidx = jnp.full((epad - e,), _N, jnp.int32)
    srcp = jnp.concatenate([edge_index[0], padidx]).reshape(_NW, nch, _K)
    dstp = jnp.concatenate([edge_index[1], padidx]).reshape(_NW, nch, _K)

    ones_mat = jnp.ones((_NP, _D), jnp.float32)
    pdeg = _scatter_call(ones_mat, srcp, dstp)
    dpt = pdeg[:, :, 0].T
    h1p, dinv = _tc1_call(x, W1, dpt)
    p1 = _scatter_call(h1p, srcp, dstp)
    h2p = _tc2_call(p1, h1p, dinv, b1[None], g1[None], be1[None], W2)
    p2 = _scatter_call(h2p, srcp, dstp)
    out = _tc3_call(p2, h2p, dinv, b2[None], g2[None], be2[None],
                    batch[:, None],
                    Wo1, bo1[None], go1[None], beo1[None],
                    Wo2, bo2[None], go2[None], beo2[None],
                    Wo3, bo3[None], Wo4, bo4[None])
    return out.reshape(-1)


# speedup vs baseline: 1.9201x; 1.9201x over previous
import jax
import jax.numpy as jnp
from jax import lax
from jax.experimental import pallas as pl
from jax.experimental.pallas import tpu as pltpu
from jax.experimental.pallas import tpu_sc as plsc

_N = 10000
_D = 128
_G = 64
_NC = 2
_NS = 16
_NW = _NC * _NS
_K = 128
_NP = 10240
_RPS = _NP // _NS


def _ceil_div(a, b):
    return -(-a // b)


def _zero_fill(ref, value):
    def fill(t, carry):
        ref[t // 8, pl.ds((t % 8) * 16, 16)] = jnp.full((16,), value, jnp.float32)
        return carry

    lax.fori_loop(0, _K * (_D // 16), fill, 0)


def _scatter_call(hp, srcp, dstp):
    nch = srcp.shape[1]

    def body(hp_hbm, src_hbm, dst_hbm, out_hbm, srcb, dstb, gb, acc, sem):
        c = lax.axis_index("c")
        s = lax.axis_index("s")
        wid = c * _NS + s

        _zero_fill(gb, 0.0)
        base = s * _RPS
        for r in range(_RPS // _K):
            pltpu.sync_copy(gb, acc.at[pl.ds(base + r * _K, _K)])

        pltpu.sync_copy(src_hbm.at[wid], srcb)
        pltpu.sync_copy(dst_hbm.at[wid], dstb)
        plsc.subcore_barrier()

        def step(j, carry):
            pltpu.async_copy(hp_hbm.at[srcb.at[j]], gb, sem).wait()
            pltpu.sync_copy(gb, acc.at[dstb.at[j]], add=True)
            return carry

        lax.fori_loop(0, nch, step, 0)
        plsc.subcore_barrier()
        pltpu.sync_copy(acc.at[pl.ds(base, _RPS)],
                        out_hbm.at[c, pl.ds(base, _RPS)])

    return pl.kernel(
        body,
        out_type=jax.ShapeDtypeStruct((_NC, _NP, _D), jnp.float32),
        mesh=plsc.VectorSubcoreMesh(core_axis_name="c", subcore_axis_name="s"),
        scratch_types=[
            pltpu.VMEM((nch, _K), jnp.int32),
            pltpu.VMEM((nch, _K), jnp.int32),
            pltpu.VMEM((_K, _D), jnp.float32),
            pltpu.VMEM_SHARED((_NP, _D), jnp.float32),
            pltpu.SemaphoreType.DMA,
        ],
    )(hp, srcp, dstp)


def _deg_call(dstp):
    nch = dstp.shape[1]

    def body(dst_hbm, out_hbm, dstb, gb, acc):
        c = lax.axis_index("c")
        s = lax.axis_index("s")
        wid = c * _NS + s

        _zero_fill(gb, 0.0)
        base = s * _RPS
        for r in range(_RPS // _K):
            pltpu.sync_copy(gb, acc.at[pl.ds(base + r * _K, _K)])

        _zero_fill(gb, 1.0)
        pltpu.sync_copy(dst_hbm.at[wid], dstb)
        plsc.subcore_barrier()

        def step(j, carry):
            pltpu.sync_copy(gb, acc.at[dstb.at[j]], add=True)
            return carry

        lax.fori_loop(0, nch, step, 0)
        plsc.subcore_barrier()
        pltpu.sync_copy(acc.at[pl.ds(base, _RPS)],
                        out_hbm.at[c, pl.ds(base, _RPS)])

    return pl.kernel(
        body,
        out_type=jax.ShapeDtypeStruct((_NC, _NP, _D), jnp.float32),
        mesh=plsc.VectorSubcoreMesh(core_axis_name="c", subcore_axis_name="s"),
        scratch_types=[
            pltpu.VMEM((nch, _K), jnp.int32),
            pltpu.VMEM((_K, _D), jnp.float32),
            pltpu.VMEM_SHARED((_NP, _D), jnp.float32),
        ],
    )(dstp)


def _rsqrt(v):
    r = lax.rsqrt(v)
    r = r * (1.5 - 0.5 * v * r * r)
    r = r * (1.5 - 0.5 * v * r * r)
    return r


def _colmean(v):
    n = v.shape[0]
    ones_row = jnp.ones((1, n), jnp.float32)
    return lax.dot_general(ones_row, v, (((1,), (0,)), ((), ())),
                           preferred_element_type=jnp.float32,
                           precision=lax.Precision.HIGHEST) * (1.0 / n)


def _bn_relu(v, gamma, beta):
    mu = _colmean(v)
    var = _colmean((v - mu) ** 2)
    return jnp.maximum((v - mu) * _rsqrt(var + 1e-5) * gamma + beta, 0.0)


def _tc1_body(x_ref, w_ref, dpt_ref, h_ref, dinv_ref):
    dpt = dpt_ref[...]
    deg = dpt[:_N, 0:1] + dpt[:_N, 1:2] + 1.0
    dinv = _rsqrt(deg)
    h = jnp.dot(x_ref[...], w_ref[...], preferred_element_type=jnp.float32)
    h_ref[pl.ds(0, _N), :] = h * dinv
    h_ref[pl.ds(_N, _NP - _N), :] = jnp.zeros((_NP - _N, _D), jnp.float32)
    dinv_ref[...] = dinv


def _tc1_call(x, W1, dpt):
    return pl.pallas_call(
        _tc1_body,
        out_shape=(jax.ShapeDtypeStruct((_NP, _D), jnp.float32),
                   jax.ShapeDtypeStruct((_N, 1), jnp.float32)),
    )(x, W1, dpt)


def _tc2_body(p_ref, hp_ref, dinv_ref, b_ref, g_ref, be_ref, w_ref, out_ref):
    tot = p_ref[0] + p_ref[1] + hp_ref[...]
    dinv = dinv_ref[...]
    conv = tot[:_N] * dinv + b_ref[...]
    a = _bn_relu(conv, g_ref[...], be_ref[...])
    h2 = jnp.dot(a, w_ref[...], preferred_element_type=jnp.float32)
    out_ref[pl.ds(0, _N), :] = h2 * dinv
    out_ref[pl.ds(_N, _NP - _N), :] = jnp.zeros((_NP - _N, _D), jnp.float32)


def _tc2_call(p, hp, dinv, b1, g1, be1, W2):
    return pl.pallas_call(
        _tc2_body,
        out_shape=jax.ShapeDtypeStruct((_NP, _D), jnp.float32),
    )(p, hp, dinv, b1, g1, be1, W2)


def _tc3_body(p_ref, hp_ref, dinv_ref, b_ref, g_ref, be_ref, bt_ref,
              w1_ref, b1_ref, g1_ref, be1_ref,
              w2_ref, b2_ref, g2_ref, be2_ref,
              w3_ref, b3_ref, w4_ref, b4_ref, out_ref):
    tot = p_ref[0] + p_ref[1] + hp_ref[...]
    conv = tot[:_N] * dinv_ref[...] + b_ref[...]
    a = _bn_relu(conv, g_ref[...], be_ref[...])
    gid = lax.broadcasted_iota(jnp.int32, (_N, _G), 1)
    onehot = (bt_ref[...] == gid).astype(jnp.float32)
    hi = lax.Precision.HIGHEST
    sums = lax.dot_general(onehot, a, (((0,), (0,)), ((), ())),
                           preferred_element_type=jnp.float32,
                           precision=hi)
    cnt = lax.dot_general(onehot, jnp.ones((_N, 1), jnp.float32),
                          (((0,), (0,)), ((), ())),
                          preferred_element_type=jnp.float32,
                          precision=hi)
    pooled = sums / jnp.maximum(cnt, 1.0)
    z = _bn_relu(jnp.dot(pooled, w1_ref[...]) + b1_ref[...],
                 g1_ref[...], be1_ref[...])
    z = _bn_relu(jnp.dot(z, w2_ref[...]) + b2_ref[...],
                 g2_ref[...], be2_ref[...])
    z = jnp.maximum(jnp.dot(z, w3_ref[...]) + b3_ref[...], 0.0)
    out_ref[...] = jnp.dot(z, w4_ref[...]) + b4_ref[...]


def _tc3_call(p, hp, dinv, b2, g2, be2, batch2d,
              Wo1, bo1, go1, beo1, Wo2, bo2, go2, beo2, Wo3, bo3, Wo4, bo4):
    return pl.pallas_call(
        _tc3_body,
        out_shape=jax.ShapeDtypeStruct((_G, 1), jnp.float32),
    )(p, hp, dinv, b2, g2, be2, batch2d,
      Wo1, bo1, go1, beo1, Wo2, bo2, go2, beo2, Wo3, bo3, Wo4, bo4)


def kernel(x, edge_index, batch,
           W1, b1, g1, be1, W2, b2, g2, be2,
           Wo1, bo1, go1, beo1, Wo2, bo2, go2, beo2, Wo3, bo3, Wo4, bo4):
    e = edge_index.shape[1]
    nch = _ceil_div(e, _NW * _K)
    epad = _NW * nch * _K
    padidx = jnp.full((epad - e,), _N, jnp.int32)
    srcp = jnp.concatenate([edge_index[0], padidx]).reshape(_NW, nch, _K)
    dstp = jnp.concatenate([edge_index[1], padidx]).reshape(_NW, nch, _K)

    pdeg = _deg_call(dstp)
    dpt = pdeg[:, :, 0].T
    h1p, dinv = _tc1_call(x, W1, dpt)
    p1 = _scatter_call(h1p, srcp, dstp)
    h2p = _tc2_call(p1, h1p, dinv, b1[None], g1[None], be1[None], W2)
    p2 = _scatter_call(h2p, srcp, dstp)
    out = _tc3_call(p2, h2p, dinv, b2[None], g2[None], be2[None],
                    batch[:, None],
                    Wo1, bo1[None], go1[None], beo1[None],
                    Wo2, bo2[None], go2[None], beo2[None],
                    Wo3, bo3[None], Wo4, bo4[None])
    return out.reshape(-1)
